# baseline (speedup 1.0000x reference)
import functools

import jax
import jax.numpy as jnp
from jax import lax
from jax.experimental import pallas as pl
from jax.experimental.pallas import tpu as pltpu
from jax.experimental.pallas import tpu_sc as plsc

BATCH = 4096
CELL = 128
_NC = 2
_NS = 16
_NW = _NC * _NS
_BPW = BATCH // _NW

_mesh = plsc.VectorSubcoreMesh(core_axis_name="c", subcore_axis_name="s")


_NCHUNK = 4
_CH = _BPW // _NCHUNK


@functools.partial(
    pl.kernel,
    mesh=_mesh,
    out_type=(
        jax.ShapeDtypeStruct((BATCH, CELL), jnp.float32),
        jax.ShapeDtypeStruct((BATCH, CELL), jnp.float32),
    ),
    scratch_types=(
        [pltpu.VMEM((_CH,), jnp.int32) for _ in range(_NCHUNK)]
        + [pltpu.VMEM((_BPW, CELL), jnp.float32),
           pltpu.VMEM((_BPW, CELL), jnp.float32)]
        + [pltpu.SemaphoreType.DMA for _ in range(2 * _NCHUNK)]
        + [pltpu.SemaphoreType.DMA]
    ),
)
def _dual_gather(idx_hbm, wh_hbm, wc_hbm, outh_hbm, outc_hbm, *scratch):
    idx_v = scratch[:_NCHUNK]
    rows_h, rows_c = scratch[_NCHUNK], scratch[_NCHUNK + 1]
    gsems = scratch[_NCHUNK + 2:_NCHUNK + 2 + 2 * _NCHUNK]
    wsem = scratch[-1]
    wid = lax.axis_index("s") * _NC + lax.axis_index("c")
    base = wid * _BPW
    for k in range(_NCHUNK):
        pltpu.sync_copy(idx_hbm.at[pl.ds(base + k * _CH, _CH)], idx_v[k])
    gathers = []
    for k in range(_NCHUNK):
        gathers.append(pltpu.async_copy(
            wh_hbm.at[idx_v[k]], rows_h.at[pl.ds(k * _CH, _CH)], gsems[2 * k]))
        gathers.append(pltpu.async_copy(
            wc_hbm.at[idx_v[k]], rows_c.at[pl.ds(k * _CH, _CH)], gsems[2 * k + 1]))
    writes = []
    for k in range(_NCHUNK):
        gathers[2 * k].wait()
        writes.append(pltpu.async_copy(
            rows_h.at[pl.ds(k * _CH, _CH)],
            outh_hbm.at[pl.ds(base + k * _CH, _CH)], wsem))
        gathers[2 * k + 1].wait()
        writes.append(pltpu.async_copy(
            rows_c.at[pl.ds(k * _CH, _CH)],
            outc_hbm.at[pl.ds(base + k * _CH, _CH)], wsem))
    for w in writes:
        w.wait()


def kernel(encoder_inputs, W_h, W_c):
    idx = encoder_inputs.reshape(-1).astype(jnp.int32)
    return _dual_gather(idx, W_h, W_c)


# speedup vs baseline: 1.0738x; 1.0738x over previous
import functools

import jax
import jax.numpy as jnp
from jax import lax
from jax.experimental import pallas as pl
from jax.experimental.pallas import tpu as pltpu
from jax.experimental.pallas import tpu_sc as plsc

BATCH = 4096
CELL = 128
_NC = 2
_NS = 16
_NW = _NC * _NS
_BPW = BATCH // _NW

_mesh = plsc.VectorSubcoreMesh(core_axis_name="c", subcore_axis_name="s")


@functools.partial(
    pl.kernel,
    mesh=_mesh,
    out_type=(
        jax.ShapeDtypeStruct((BATCH, CELL), jnp.float32),
        jax.ShapeDtypeStruct((BATCH, CELL), jnp.float32),
    ),
    scratch_types=[
        pltpu.VMEM((_BPW,), jnp.int32),
        pltpu.VMEM((_BPW, CELL), jnp.float32),
        pltpu.VMEM((_BPW, CELL), jnp.float32),
        pltpu.SemaphoreType.DMA,
        pltpu.SemaphoreType.DMA,
        pltpu.SemaphoreType.DMA,
    ],
)
def _dual_gather(idx_hbm, wh_hbm, wc_hbm, outh_hbm, outc_hbm,
                 idx_v, rows_h, rows_c, sem_h, sem_c, sem_w):
    wid = lax.axis_index("s") * _NC + lax.axis_index("c")
    base = wid * _BPW
    pltpu.sync_copy(idx_hbm.at[pl.ds(base, _BPW)], idx_v)
    cp_h = pltpu.async_copy(wh_hbm.at[idx_v], rows_h, sem_h)
    cp_c = pltpu.async_copy(wc_hbm.at[idx_v], rows_c, sem_c)
    cp_h.wait()
    w_h = pltpu.async_copy(rows_h, outh_hbm.at[pl.ds(base, _BPW)], sem_w)
    cp_c.wait()
    w_c = pltpu.async_copy(rows_c, outc_hbm.at[pl.ds(base, _BPW)], sem_w)
    w_h.wait()
    w_c.wait()


def kernel(encoder_inputs, W_h, W_c):
    idx = encoder_inputs.reshape(-1).astype(jnp.int32)
    return _dual_gather(idx, W_h, W_c)
